# baseline (speedup 1.0000x reference)
import functools

import jax
import jax.numpy as jnp
from jax.experimental import pallas as pl

N = 32768
D = 768
E = 8
M = 8


def _moe_body(x1_ref, x2_ref, wcat_ref, gate_b_ref, out_ref):
    _IMIN = jnp.int32(-(2 ** 31))
    w = wcat_ref[...]
    y = (jnp.dot(x1_ref[...], w[:D // 2],
                 preferred_element_type=jnp.float32)
         + jnp.dot(x2_ref[...], w[D // 2:],
                   preferred_element_type=jnp.float32))
    g = y[:, :E] + gate_b_ref[...]
    aeo = y[:, E:]

    gt = jnp.swapaxes(g, 0, 1)
    sub_e = jax.lax.broadcasted_iota(jnp.int32, gt.shape, 0)
    gb = jax.lax.bitcast_convert_type(gt, jnp.int32)
    key = gb ^ ((gb >> 31) & jnp.int32(0x7FFFFFFF))
    kk = (key & jnp.int32(-8)) | (jnp.int32(E - 1) - sub_e)
    k1 = jnp.max(kk, axis=0, keepdims=True)
    oh1 = kk == k1
    kk2 = jnp.where(oh1, _IMIN, kk)
    k2 = jnp.max(kk2, axis=0, keepdims=True)
    oh2 = kk2 == k2

    def _dec(k):
        return jax.lax.bitcast_convert_type(
            k ^ ((k >> 31) & jnp.int32(0x7FFFFFFF)), jnp.float32)

    t = jnp.exp(_dec(k2) - _dec(k1))
    w1 = 1.0 / (1.0 + t)
    w2 = t * w1
    w8t = jnp.where(oh1, w1, 0.0) + jnp.where(oh2, w2, 0.0)
    w8 = jnp.swapaxes(w8t, 0, 1)

    si = jax.lax.broadcasted_iota(jnp.int32, (E, E * M), 0)
    sj = jax.lax.broadcasted_iota(jnp.int32, (E, E * M), 1)
    s = (si == (sj >> 3)).astype(jnp.bfloat16)
    ri = jax.lax.broadcasted_iota(jnp.int32, (E * M, M), 0)
    rj = jax.lax.broadcasted_iota(jnp.int32, (E * M, M), 1)
    r = ((ri & jnp.int32(M - 1)) == rj).astype(jnp.bfloat16)
    w_em = jnp.dot(w8.astype(jnp.bfloat16), s,
                   preferred_element_type=jnp.float32)
    out_ref[...] = jnp.dot((aeo * w_em).astype(jnp.bfloat16), r,
                           preferred_element_type=jnp.float32)


@functools.partial(jax.jit, static_argnames=("block_n",))
def _moe_forward(x, wcat, gate_b2d, block_n=2048):
    grid = (N // block_n,)
    return pl.pallas_call(
        _moe_body,
        grid=grid,
        in_specs=[
            pl.BlockSpec((block_n, D // 2), lambda i: (i, 0)),
            pl.BlockSpec((block_n, D // 2), lambda i: (i, 1)),
            pl.BlockSpec((D, E + E * M), lambda i: (0, 0)),
            pl.BlockSpec((1, E), lambda i: (0, 0)),
        ],
        out_specs=pl.BlockSpec((block_n, M), lambda i: (i, 0)),
        out_shape=jax.ShapeDtypeStruct((N, M), jnp.float32),
    )(x, x, wcat, gate_b2d)


def kernel(x, gate_W, gate_b, expert_W):
    w_experts = jnp.transpose(expert_W, (1, 0, 2)).reshape(D, E * M)
    wcat = jnp.concatenate([gate_W, w_experts], axis=1)
    return _moe_forward(x, wcat, gate_b.reshape(1, E))


# speedup vs baseline: 1.0619x; 1.0619x over previous
import functools

import jax
import jax.numpy as jnp
from jax.experimental import pallas as pl

N = 32768
D = 768
E = 8
M = 8


def _moe_body(x1_ref, x2_ref, wcat_ref, gate_b_ref, out_ref):
    _IMIN = jnp.int32(-(2 ** 31))
    w = wcat_ref[...]
    y = (jnp.dot(x1_ref[...], w[:D // 2],
                 preferred_element_type=jnp.float32)
         + jnp.dot(x2_ref[...], w[D // 2:],
                   preferred_element_type=jnp.float32))
    aeo = y[:, :E * M]
    g = y[:, E * M:] + gate_b_ref[...]

    gt = jnp.swapaxes(g, 0, 1)
    sub_e = jax.lax.broadcasted_iota(jnp.int32, gt.shape, 0)
    gb = jax.lax.bitcast_convert_type(gt, jnp.int32)
    key = gb ^ ((gb >> 31) & jnp.int32(0x7FFFFFFF))
    kk = (key & jnp.int32(-8)) | (jnp.int32(E - 1) - sub_e)
    k1 = jnp.max(kk, axis=0, keepdims=True)
    oh1 = kk == k1
    kk2 = jnp.where(oh1, _IMIN, kk)
    k2 = jnp.max(kk2, axis=0, keepdims=True)
    oh2 = kk2 == k2

    def _dec(k):
        return jax.lax.bitcast_convert_type(
            k ^ ((k >> 31) & jnp.int32(0x7FFFFFFF)), jnp.float32)

    t = jnp.exp(_dec(k2) - _dec(k1))
    w1 = 1.0 / (1.0 + t)
    w2 = t * w1
    w8t = jnp.where(oh1, w1, 0.0) + jnp.where(oh2, w2, 0.0)
    w8 = jnp.swapaxes(w8t, 0, 1)

    si = jax.lax.broadcasted_iota(jnp.int32, (E, E * M), 0)
    sj = jax.lax.broadcasted_iota(jnp.int32, (E, E * M), 1)
    s = (si == (sj >> 3)).astype(jnp.bfloat16)
    ri = jax.lax.broadcasted_iota(jnp.int32, (E * M, M), 0)
    rj = jax.lax.broadcasted_iota(jnp.int32, (E * M, M), 1)
    r = ((ri & jnp.int32(M - 1)) == rj).astype(jnp.bfloat16)
    w_em = jnp.dot(w8.astype(jnp.bfloat16), s,
                   preferred_element_type=jnp.float32)
    out_ref[...] = jnp.dot((aeo * w_em).astype(jnp.bfloat16), r,
                           preferred_element_type=jnp.float32)


@functools.partial(jax.jit, static_argnames=("block_n",))
def _moe_forward(x, wcat, gate_b2d, block_n=4096):
    grid = (N // block_n,)
    return pl.pallas_call(
        _moe_body,
        grid=grid,
        in_specs=[
            pl.BlockSpec((block_n, D // 2), lambda i: (i, 0)),
            pl.BlockSpec((block_n, D // 2), lambda i: (i, 1)),
            pl.BlockSpec((D, E + E * M), lambda i: (0, 0)),
            pl.BlockSpec((1, E), lambda i: (0, 0)),
        ],
        out_specs=pl.BlockSpec((block_n, M), lambda i: (i, 0)),
        out_shape=jax.ShapeDtypeStruct((N, M), jnp.float32),
    )(x, x, wcat, gate_b2d)


def kernel(x, gate_W, gate_b, expert_W):
    w_experts = jnp.transpose(expert_W, (1, 0, 2)).reshape(D, E * M)
    wcat = jnp.concatenate([w_experts, gate_W], axis=1)
    return _moe_forward(x, wcat, gate_b.reshape(1, E))


# speedup vs baseline: 1.1285x; 1.0627x over previous
import functools

import jax
import jax.numpy as jnp
from jax.experimental import pallas as pl

N = 32768
D = 768
E = 8
M = 8


def _moe_body(x1_ref, x2_ref, wcat_ref, gate_b_ref, out_ref):
    _IMIN = jnp.int32(-(2 ** 31))
    w = wcat_ref[...]
    y = (jnp.dot(x1_ref[...], w[:D // 2],
                 preferred_element_type=jnp.float32)
         + jnp.dot(x2_ref[...], w[D // 2:],
                   preferred_element_type=jnp.float32))
    g = y[:, :E] + gate_b_ref[...]

    gt = jnp.swapaxes(g, 0, 1)
    sub_e = jax.lax.broadcasted_iota(jnp.int32, gt.shape, 0)
    gb = jax.lax.bitcast_convert_type(gt, jnp.int32)
    key = gb ^ ((gb >> 31) & jnp.int32(0x7FFFFFFF))
    kk = (key & jnp.int32(-8)) | (jnp.int32(E - 1) - sub_e)
    k1 = jnp.max(kk, axis=0, keepdims=True)
    oh1 = kk == k1
    kk2 = jnp.where(oh1, _IMIN, kk)
    k2 = jnp.max(kk2, axis=0, keepdims=True)
    oh2 = kk2 == k2

    def _dec(k):
        return jax.lax.bitcast_convert_type(
            k ^ ((k >> 31) & jnp.int32(0x7FFFFFFF)), jnp.float32)

    t = jnp.exp(_dec(k2) - _dec(k1))
    w1 = 1.0 / (1.0 + t)
    w2 = t * w1
    w8t = jnp.where(oh1, w1, 0.0) + jnp.where(oh2, w2, 0.0)
    w8 = jnp.swapaxes(w8t, 0, 1)

    C = E + E * M
    si = jax.lax.broadcasted_iota(jnp.int32, (E, C), 0)
    sj = jax.lax.broadcasted_iota(jnp.int32, (E, C), 1)
    s = ((si == ((sj - E) >> 3)) & (sj >= E)).astype(jnp.bfloat16)
    ri = jax.lax.broadcasted_iota(jnp.int32, (C, M), 0)
    rj = jax.lax.broadcasted_iota(jnp.int32, (C, M), 1)
    r = ((((ri - E) & jnp.int32(M - 1)) == rj)
         & (ri >= E)).astype(jnp.bfloat16)
    w_em = jnp.dot(w8.astype(jnp.bfloat16), s,
                   preferred_element_type=jnp.float32)
    out_ref[...] = jnp.dot((y * w_em).astype(jnp.bfloat16), r,
                           preferred_element_type=jnp.float32)


@functools.partial(jax.jit, static_argnames=("block_n",))
def _moe_forward(x, wcat, gate_b2d, block_n=4096):
    grid = (N // block_n,)
    return pl.pallas_call(
        _moe_body,
        grid=grid,
        in_specs=[
            pl.BlockSpec((block_n, D // 2), lambda i: (i, 0)),
            pl.BlockSpec((block_n, D // 2), lambda i: (i, 1)),
            pl.BlockSpec((D, E + E * M), lambda i: (0, 0)),
            pl.BlockSpec((1, E), lambda i: (0, 0)),
        ],
        out_specs=pl.BlockSpec((block_n, M), lambda i: (i, 0)),
        out_shape=jax.ShapeDtypeStruct((N, M), jnp.float32),
    )(x, x, wcat, gate_b2d)


def kernel(x, gate_W, gate_b, expert_W):
    w_experts = jnp.transpose(expert_W, (1, 0, 2)).reshape(D, E * M)
    wcat = jnp.concatenate([gate_W, w_experts], axis=1)
    return _moe_forward(x, wcat, gate_b.reshape(1, E))
